# baseline (speedup 1.0000x reference)
import functools

import jax
import jax.numpy as jnp
from jax import lax
from jax.experimental import pallas as pl
from jax.experimental.pallas import tpu as pltpu
from jax.experimental.pallas import tpu_sc as plsc


def _make_copy_kernel(L, D, dtype, num_cores):
    rows_per_c = L // num_cores
    chunk = 64
    nbuf = 24
    look = 12
    nchunks = rows_per_c // chunk

    mesh = plsc.ScalarSubcoreMesh(axis_name="c", num_cores=num_cores)

    @functools.partial(
        pl.kernel,
        mesh=mesh,
        out_type=jax.ShapeDtypeStruct((1, L, D), dtype),
        scratch_types=[
            pltpu.VMEM_SHARED((nbuf, chunk, D), dtype),
            pltpu.SemaphoreType.DMA,
            pltpu.SemaphoreType.DMA,
        ],
    )
    def copy_k(table_hbm, out_hbm, buf, in_sem, out_sem):
        base = lax.axis_index("c") * rows_per_c

        def load(j):
            pltpu.async_copy(
                table_hbm.at[pl.ds(base + j * chunk, chunk)],
                buf.at[j % nbuf],
                in_sem,
            )

        def store(j):
            pltpu.async_copy(
                buf.at[j % nbuf],
                out_hbm.at[0, pl.ds(base + j * chunk, chunk)],
                out_sem,
            )

        def drain_in(j):
            pltpu.make_async_copy(
                table_hbm.at[pl.ds(base, chunk)], buf.at[j % nbuf], in_sem
            ).wait()

        def drain_out(j):
            pltpu.make_async_copy(
                buf.at[j % nbuf], out_hbm.at[0, pl.ds(base, chunk)], out_sem
            ).wait()

        for j in range(min(look, nchunks)):
            load(j)
        for i in range(nchunks):
            d = i - (nbuf - look)
            if d >= 0:
                drain_out(d)
            j = i + look
            if j < nchunks:
                load(j)
            drain_in(i)
            store(i)
        for d in range(max(0, nchunks - (nbuf - look)), nchunks):
            drain_out(d)

    return copy_k


def kernel(hidden_states, position_table):
    L = hidden_states.shape[1]
    D = position_table.shape[1]
    copy_k = _make_copy_kernel(L, D, position_table.dtype, 2)
    return copy_k(position_table)


# speedup vs baseline: 1.0349x; 1.0349x over previous
import functools

import jax
import jax.numpy as jnp
from jax import lax
from jax.experimental import pallas as pl
from jax.experimental.pallas import tpu as pltpu
from jax.experimental.pallas import tpu_sc as plsc


def _make_copy_kernel(L, D, dtype, num_cores):
    rows_per_c = L // num_cores
    chunk = 32
    nbuf = 32
    look = 16
    nchunks = rows_per_c // chunk

    mesh = plsc.ScalarSubcoreMesh(axis_name="c", num_cores=num_cores)

    @functools.partial(
        pl.kernel,
        mesh=mesh,
        out_type=jax.ShapeDtypeStruct((1, L, D), dtype),
        scratch_types=[
            pltpu.VMEM_SHARED((nbuf, chunk, D), dtype),
            pltpu.SemaphoreType.DMA,
            pltpu.SemaphoreType.DMA,
        ],
    )
    def copy_k(table_hbm, out_hbm, buf, in_sem, out_sem):
        base = lax.axis_index("c") * rows_per_c

        def load(j):
            pltpu.async_copy(
                table_hbm.at[pl.ds(base + j * chunk, chunk)],
                buf.at[j % nbuf],
                in_sem,
            )

        def store(j):
            pltpu.async_copy(
                buf.at[j % nbuf],
                out_hbm.at[0, pl.ds(base + j * chunk, chunk)],
                out_sem,
            )

        def drain_in(j):
            pltpu.make_async_copy(
                table_hbm.at[pl.ds(base, chunk)], buf.at[j % nbuf], in_sem
            ).wait()

        def drain_out(j):
            pltpu.make_async_copy(
                buf.at[j % nbuf], out_hbm.at[0, pl.ds(base, chunk)], out_sem
            ).wait()

        for j in range(min(look, nchunks)):
            load(j)
        for i in range(nchunks):
            d = i - (nbuf - look)
            if d >= 0:
                drain_out(d)
            j = i + look
            if j < nchunks:
                load(j)
            drain_in(i)
            store(i)
        for d in range(max(0, nchunks - (nbuf - look)), nchunks):
            drain_out(d)

    return copy_k


def kernel(hidden_states, position_table):
    L = hidden_states.shape[1]
    D = position_table.shape[1]
    copy_k = _make_copy_kernel(L, D, position_table.dtype, 2)
    return copy_k(position_table)
